# baseline (speedup 1.0000x reference)
import functools

import jax
import jax.numpy as jnp
from jax import lax
from jax.experimental import pallas as pl
from jax.experimental.pallas import tpu as pltpu
from jax.experimental.pallas import tpu_sc as plsc

N = 8192
D = 128
N_WORKERS = 32
ROWS_W = N // N_WORKERS
B = 128
SB = 64


def _newton_sqrt(q):
    qi = lax.bitcast_convert_type(q, jnp.int32)
    yi = 0x5F3759DF - (qi >> 1)
    y = lax.bitcast_convert_type(yi, jnp.float32)
    for _ in range(3):
        y = y * (1.5 - 0.5 * q * y * y)
    return q * y


@functools.lru_cache(maxsize=None)
def _make_kernel(L: int):
    NBLK = L // B
    mesh = plsc.VectorSubcoreMesh(core_axis_name="c", subcore_axis_name="s")

    @functools.partial(
        pl.kernel,
        mesh=mesh,
        compiler_params=pltpu.CompilerParams(needs_layout_passes=False, use_tc_tiling_on_sc=False),
        out_type=jax.ShapeDtypeStruct((N_WORKERS, 16), jnp.float32),
        scratch_types=[
            pltpu.VMEM((ROWS_W, D // 2), jnp.int32),
            pltpu.VMEM((SB * B,), jnp.int32),
            pltpu.VMEM((SB * B,), jnp.int32),
            pltpu.VMEM((SB * B,), jnp.float32),
            pltpu.VMEM((B, D // 2), jnp.int32),
            pltpu.VMEM((B, D // 2), jnp.int32),
            pltpu.VMEM((16 * 17,), jnp.float32),
            pltpu.VMEM((16,), jnp.int32),
            pltpu.VMEM((16,), jnp.float32),
            pltpu.SemaphoreType.DMA,
            pltpu.SemaphoreType.DMA,
            pltpu.SemaphoreType.DMA,
            pltpu.SemaphoreType.DMA,
        ],
    )
    def k(x_hbm, idx0_hbm, idx1_hbm, s_hbm, bnd_hbm, out_hbm,
          xi_v, idx0_v, idx1_v, s_v, rjA, rjB, tb_v, bnd_v, loss_v,
          semA, semA2, semB, semB2):
        cid = lax.axis_index("c")
        sid = lax.axis_index("s")
        wid = sid * 2 + cid
        base_node = wid * ROWS_W

        pltpu.sync_copy(bnd_hbm.at[wid], bnd_v)
        bnd = bnd_v[...]
        lo = bnd[0]
        hi = bnd[1]
        blk0 = lo // B
        blk_end = (hi + B - 1) // B

        pltpu.sync_copy(x_hbm.at[pl.ds(base_node, ROWS_W)], xi_v)

        lane = lax.iota(jnp.int32, 16)
        t_idx0 = lane * 17

        H = B // 2

        def issue(local_b, rj, sem, sem2):
            lb = jnp.where(local_b >= SB, 0, local_b)
            pltpu.async_copy(
                x_hbm.at[idx1_v.at[pl.ds(lb * B, H)]], rj.at[pl.ds(0, H)], sem)
            pltpu.async_copy(
                x_hbm.at[idx1_v.at[pl.ds(lb * B + H, H)]], rj.at[pl.ds(H, H)], sem2)

        def drain(rj, sem, sem2):
            pltpu.make_async_copy(
                x_hbm.at[idx1_v.at[pl.ds(0, H)]], rj.at[pl.ds(0, H)], sem).wait()
            pltpu.make_async_copy(
                x_hbm.at[idx1_v.at[pl.ds(0, H)]], rj.at[pl.ds(H, H)], sem2).wait()

        def compute(rj, cblk0, local_b, loss16):
            gb = cblk0 + local_b
            eb = gb * B

            def grp(g, loss16):
                soff = local_b * B + g * 16
                sv = s_v[pl.ds(soff, 16)]
                iv = idx0_v[pl.ds(soff, 16)]
                for h in range(2):
                    accs = []
                    for l in range(h * 8, h * 8 + 8):
                        il = iv[l] - base_node
                        il = jnp.minimum(jnp.maximum(il, 0), ROWS_W - 1)
                        e = g * 16 + l
                        a0 = None
                        a1 = None
                        for c in range(D // 32):
                            bi = plsc.bitcast(xi_v[il, pl.ds(c * 16, 16)], jnp.bfloat16)
                            bj = plsc.bitcast(rj[e, pl.ds(c * 16, 16)], jnp.bfloat16)
                            tc = bi - bj
                            u, v = plsc.unpack(tc, format=plsc.PackFormat.INTERLEAVED)
                            if a0 is None:
                                a0 = u * u
                                a1 = v * v
                            else:
                                a0 = a0 + u * u
                                a1 = a1 + v * v
                        accs.append(a0 + a1)
                    for l in range(8):
                        tb_v[pl.ds((h * 8 + l) * 17, 16)] = accs[l]
                q = plsc.load_gather(tb_v, [t_idx0])
                for f in range(1, 16):
                    q = q + plsc.load_gather(tb_v, [t_idx0 + f])
                d = _newton_sqrt(q)
                t = sv - d
                e16 = eb + g * 16 + lane
                m = jnp.logical_and(e16 >= lo, e16 < hi)
                return loss16 + jnp.where(m, t * t, 0.0)

            return lax.fori_loop(0, B // 16, grp, loss16)

        def chunk_body(c, loss16):
            cblk0 = blk0 + c * SB
            soff = cblk0 * B
            pltpu.sync_copy(idx0_hbm.at[pl.ds(soff, SB * B)], idx0_v)
            pltpu.sync_copy(idx1_hbm.at[pl.ds(soff, SB * B)], idx1_v)
            pltpu.sync_copy(s_hbm.at[pl.ds(soff, SB * B)], s_v)
            npairs = jnp.minimum(SB, blk_end - cblk0)
            npairs = (npairs + 1) // 2

            issue(0, rjA, semA, semA2)

            def pair(p, loss16):
                a = 2 * p
                drain(rjA, semA, semA2)
                issue(a + 1, rjB, semB, semB2)
                loss16 = compute(rjA, cblk0, a, loss16)
                drain(rjB, semB, semB2)
                issue(a + 2, rjA, semA, semA2)
                loss16 = compute(rjB, cblk0, a + 1, loss16)
                return loss16

            loss16 = lax.fori_loop(0, npairs, pair, loss16)
            drain(rjA, semA, semA2)
            return loss16

        nchunks = (blk_end - blk0 + SB - 1) // SB
        loss16 = lax.fori_loop(
            0, nchunks, chunk_body, jnp.zeros((16,), jnp.float32))
        loss_v[...] = loss16
        pltpu.sync_copy(loss_v, out_hbm.at[wid])

    return k


def kernel(input, small_dists, indices):
    E = indices.shape[0]
    L = -(-E // B) * B + SB * B
    pad = L - E
    idx0 = jnp.pad(indices[:, 0], (0, pad))
    idx1 = jnp.pad(indices[:, 1], (0, pad))
    s = jnp.pad(small_dists, (0, pad))
    cuts = jnp.arange(N_WORKERS + 1, dtype=jnp.int32) * ROWS_W
    b = jnp.searchsorted(indices[:, 0], cuts, side="left").astype(jnp.int32)
    bnd = jnp.zeros((N_WORKERS, 16), jnp.int32)
    bnd = bnd.at[:, 0].set(b[:-1]).at[:, 1].set(b[1:])
    xbf = input.astype(jnp.bfloat16)
    xpk = lax.bitcast_convert_type(xbf.reshape(N, D // 2, 2), jnp.int32)
    out = _make_kernel(L)(xpk, idx0, idx1, s, bnd)
    return out.sum()


# speedup vs baseline: 1.0086x; 1.0086x over previous
import functools

import jax
import jax.numpy as jnp
from jax import lax
from jax.experimental import pallas as pl
from jax.experimental.pallas import tpu as pltpu
from jax.experimental.pallas import tpu_sc as plsc

N = 8192
D = 128
N_WORKERS = 32
ROWS_W = N // N_WORKERS
B = 128
SB = 64


def _newton_sqrt(q):
    qi = lax.bitcast_convert_type(q, jnp.int32)
    yi = 0x5F3759DF - (qi >> 1)
    y = lax.bitcast_convert_type(yi, jnp.float32)
    for _ in range(3):
        y = y * (1.5 - 0.5 * q * y * y)
    return q * y


@functools.lru_cache(maxsize=None)
def _make_kernel(L: int):
    NBLK = L // B
    mesh = plsc.VectorSubcoreMesh(core_axis_name="c", subcore_axis_name="s")

    @functools.partial(
        pl.kernel,
        mesh=mesh,
        compiler_params=pltpu.CompilerParams(needs_layout_passes=False, use_tc_tiling_on_sc=False),
        out_type=jax.ShapeDtypeStruct((N_WORKERS, 16), jnp.float32),
        scratch_types=[
            pltpu.VMEM((ROWS_W, D // 2), jnp.int32),
            pltpu.VMEM((SB * B,), jnp.int32),
            pltpu.VMEM((SB * B,), jnp.int32),
            pltpu.VMEM((SB * B,), jnp.float32),
            pltpu.VMEM((B, D // 2), jnp.int32),
            pltpu.VMEM((B, D // 2), jnp.int32),
            pltpu.VMEM((16 * 17,), jnp.float32),
            pltpu.VMEM((16,), jnp.int32),
            pltpu.VMEM((16,), jnp.float32),
            pltpu.SemaphoreType.DMA,
            pltpu.SemaphoreType.DMA,
        ],
    )
    def k(x_hbm, idx0_hbm, idx1_hbm, s_hbm, bnd_hbm, out_hbm,
          xi_v, idx0_v, idx1_v, s_v, rjA, rjB, tb_v, bnd_v, loss_v,
          semA, semB):
        cid = lax.axis_index("c")
        sid = lax.axis_index("s")
        wid = sid * 2 + cid
        base_node = wid * ROWS_W

        cp_xi = pltpu.async_copy(
            x_hbm.at[pl.ds(base_node, ROWS_W)], xi_v, semB)
        pltpu.sync_copy(bnd_hbm.at[wid], bnd_v)
        bnd = bnd_v[...]
        lo = bnd[0]
        hi = bnd[1]
        blk0 = lo // B
        blk_end = (hi + B - 1) // B
        cp_xi.wait()

        lane = lax.iota(jnp.int32, 16)
        t_idx0 = lane * 17

        def issue(local_b, rj, sem):
            lb = jnp.where(local_b >= SB, 0, local_b)
            pltpu.async_copy(
                x_hbm.at[idx1_v.at[pl.ds(lb * B, B)]], rj, sem)

        def drain(rj, sem):
            pltpu.make_async_copy(
                x_hbm.at[idx1_v.at[pl.ds(0, B)]], rj, sem).wait()

        def compute(rj, cblk0, local_b, loss16):
            gb = cblk0 + local_b
            eb = gb * B

            def grp(g, loss16):
                soff = local_b * B + g * 16
                sv = s_v[pl.ds(soff, 16)]
                iv = idx0_v[pl.ds(soff, 16)]
                for h in range(2):
                    accs = []
                    for l in range(h * 8, h * 8 + 8):
                        il = iv[l] - base_node
                        il = jnp.minimum(jnp.maximum(il, 0), ROWS_W - 1)
                        e = g * 16 + l
                        a0 = None
                        a1 = None
                        for c in range(D // 32):
                            bi = plsc.bitcast(xi_v[il, pl.ds(c * 16, 16)], jnp.bfloat16)
                            bj = plsc.bitcast(rj[e, pl.ds(c * 16, 16)], jnp.bfloat16)
                            tc = bi - bj
                            u, v = plsc.unpack(tc, format=plsc.PackFormat.INTERLEAVED)
                            if a0 is None:
                                a0 = u * u
                                a1 = v * v
                            else:
                                a0 = a0 + u * u
                                a1 = a1 + v * v
                        accs.append(a0 + a1)
                    for l in range(8):
                        tb_v[pl.ds((h * 8 + l) * 17, 16)] = accs[l]
                q = plsc.load_gather(tb_v, [t_idx0])
                for f in range(1, 16):
                    q = q + plsc.load_gather(tb_v, [t_idx0 + f])
                d = _newton_sqrt(q)
                t = sv - d
                e16 = eb + g * 16 + lane
                m = jnp.logical_and(e16 >= lo, e16 < hi)
                return loss16 + jnp.where(m, t * t, 0.0)

            return lax.fori_loop(0, B // 16, grp, loss16)

        def chunk_body(c, loss16):
            cblk0 = blk0 + c * SB
            soff = cblk0 * B
            pltpu.sync_copy(idx1_hbm.at[pl.ds(soff, SB * B)], idx1_v)
            issue(0, rjA, semA)
            pltpu.sync_copy(idx0_hbm.at[pl.ds(soff, SB * B)], idx0_v)
            pltpu.sync_copy(s_hbm.at[pl.ds(soff, SB * B)], s_v)
            npairs = jnp.minimum(SB, blk_end - cblk0)
            npairs = (npairs + 1) // 2

            def pair(p, loss16):
                a = 2 * p
                drain(rjA, semA)
                issue(a + 1, rjB, semB)
                loss16 = compute(rjA, cblk0, a, loss16)
                drain(rjB, semB)
                issue(a + 2, rjA, semA)
                loss16 = compute(rjB, cblk0, a + 1, loss16)
                return loss16

            loss16 = lax.fori_loop(0, npairs, pair, loss16)
            drain(rjA, semA)
            return loss16

        nchunks = (blk_end - blk0 + SB - 1) // SB
        loss16 = lax.fori_loop(
            0, nchunks, chunk_body, jnp.zeros((16,), jnp.float32))
        loss_v[...] = loss16
        pltpu.sync_copy(loss_v, out_hbm.at[wid])

    return k


def kernel(input, small_dists, indices):
    E = indices.shape[0]
    L = -(-E // B) * B + SB * B
    pad = L - E
    idx0 = jnp.pad(indices[:, 0], (0, pad))
    idx1 = jnp.pad(indices[:, 1], (0, pad))
    s = jnp.pad(small_dists, (0, pad))
    cuts = jnp.arange(N_WORKERS + 1, dtype=jnp.int32) * ROWS_W
    b = jnp.searchsorted(indices[:, 0], cuts, side="left").astype(jnp.int32)
    bnd = jnp.zeros((N_WORKERS, 16), jnp.int32)
    bnd = bnd.at[:, 0].set(b[:-1]).at[:, 1].set(b[1:])
    xbf = input.astype(jnp.bfloat16)
    xpk = lax.bitcast_convert_type(xbf.reshape(N, D // 2, 2), jnp.int32)
    out = _make_kernel(L)(xpk, idx0, idx1, s, bnd)
    return out.sum()
